# baseline (speedup 1.0000x reference)
import jax
import jax.numpy as jnp
from jax import lax
from jax.experimental import pallas as pl
from jax.experimental.pallas import tpu as pltpu
from jax.experimental.pallas import tpu_sc as plsc

VOCAB = 10
DIM = 4
BATCH = 16384
NC = 2
NS = 16
L = 16
NW = NC * NS
B_PER_W = BATCH // NW

_EMB_OFF = 0
_FCW_OFF = 40
_FCB_OFF = 48


def _full(val):
    return jnp.full((L,), val, jnp.int32)


def _body(w1_hbm, w2_hbm, embf_hbm, fcw_hbm, fcb_hbm, out_hbm,
          w1_v, w2_v, out_v, params_v, tab_v, sem_idx, sem_par):
    wid = lax.axis_index("s") * NC + lax.axis_index("c")
    base = wid * B_PER_W

    c1 = pltpu.async_copy(w1_hbm.at[pl.ds(base, B_PER_W)], w1_v, sem_idx)
    c2 = pltpu.async_copy(w2_hbm.at[pl.ds(base, B_PER_W)], w2_v, sem_idx)
    c3 = pltpu.async_copy(embf_hbm, params_v.at[pl.ds(_EMB_OFF, VOCAB * DIM)],
                          sem_par)
    c4 = pltpu.async_copy(fcw_hbm, params_v.at[pl.ds(_FCW_OFF, 2 * DIM)],
                          sem_par)
    c5 = pltpu.async_copy(fcb_hbm, params_v.at[pl.ds(_FCB_OFF, 1)], sem_par)
    c3.wait()
    c4.wait()
    c5.wait()

    lanes = lax.iota(jnp.int32, L)
    vrow = jnp.minimum(lanes, VOCAB - 1)
    fcwb = params_v[pl.ds(_FCW_OFF, L)]
    s1 = jnp.take_along_axis(fcwb, _full(2 * DIM), axis=0)
    s2 = jnp.zeros((L,), jnp.float32)
    for d in range(DIM):
        col = plsc.load_gather(params_v, [vrow * DIM + d])
        wa = jnp.take_along_axis(fcwb, _full(d), axis=0)
        wb = jnp.take_along_axis(fcwb, _full(DIM + d), axis=0)
        s1 = s1 + col * wa
        s2 = s2 + col * wb
    s1n = -s1
    s2n = -s2

    for v1 in range(VOCAB):
        b1 = jnp.take_along_axis(s1n, _full(v1), axis=0)
        tab_v[pl.ds(v1 * L, L)] = 1.0 / (1.0 + jnp.exp(b1 + s2n))

    c1.wait()
    c2.wait()

    half = B_PER_W // 2
    for j in range(half // L):
        sl = pl.ds(j * L, L)
        idx = (w1_v[sl] << 4) + w2_v[sl]
        out_v[sl] = plsc.load_gather(tab_v, [idx])
    o1 = pltpu.async_copy(out_v.at[pl.ds(0, half)],
                          out_hbm.at[pl.ds(base, half)], sem_idx)
    for j in range(half // L, B_PER_W // L):
        sl = pl.ds(j * L, L)
        idx = (w1_v[sl] << 4) + w2_v[sl]
        out_v[sl] = plsc.load_gather(tab_v, [idx])
    o2 = pltpu.async_copy(out_v.at[pl.ds(half, half)],
                          out_hbm.at[pl.ds(base + half, half)], sem_idx)
    o1.wait()
    o2.wait()


@jax.jit
def kernel(w1, w2, embedding, fc_w, fc_b):
    mesh = plsc.VectorSubcoreMesh(core_axis_name="c", subcore_axis_name="s",
                                  num_cores=NC, num_subcores=NS)
    run = pl.kernel(
        _body,
        out_type=jax.ShapeDtypeStruct((BATCH,), jnp.float32),
        mesh=mesh,
        scratch_types=[
            pltpu.VMEM((B_PER_W,), jnp.int32),
            pltpu.VMEM((B_PER_W,), jnp.int32),
            pltpu.VMEM((B_PER_W,), jnp.float32),
            pltpu.VMEM((128,), jnp.float32),
            pltpu.VMEM((VOCAB * L,), jnp.float32),
            pltpu.SemaphoreType.DMA,
            pltpu.SemaphoreType.DMA,
        ],
        compiler_params=pltpu.CompilerParams(needs_layout_passes=False),
        name="similarity_sc",
    )
    return run(w1, w2, embedding.reshape(-1), fc_w.reshape(-1), fc_b)


# speedup vs baseline: 1.0084x; 1.0084x over previous
import jax
import jax.numpy as jnp
from jax import lax
from jax.experimental import pallas as pl
from jax.experimental.pallas import tpu as pltpu
from jax.experimental.pallas import tpu_sc as plsc

VOCAB = 10
DIM = 4
BATCH = 16384
NC = 2
NS = 16
L = 16
NW = NC * NS
B_PER_W = BATCH // NW

_EMB_OFF = 0
_FCW_OFF = 40
_FCB_OFF = 48


def _full(val):
    return jnp.full((L,), val, jnp.int32)


def _body(w1_hbm, w2_hbm, embf_hbm, fcw_hbm, fcb_hbm, out_hbm,
          w1_v, w2_v, out_v, params_v, tab_v, sem_idx, sem_par):
    wid = lax.axis_index("s") * NC + lax.axis_index("c")
    base = wid * B_PER_W

    c1 = pltpu.async_copy(w1_hbm.at[pl.ds(base, B_PER_W)], w1_v, sem_idx)
    c2 = pltpu.async_copy(w2_hbm.at[pl.ds(base, B_PER_W)], w2_v, sem_idx)
    c3 = pltpu.async_copy(embf_hbm, params_v.at[pl.ds(_EMB_OFF, VOCAB * DIM)],
                          sem_par)
    c4 = pltpu.async_copy(fcw_hbm, params_v.at[pl.ds(_FCW_OFF, 2 * DIM)],
                          sem_par)
    c5 = pltpu.async_copy(fcb_hbm, params_v.at[pl.ds(_FCB_OFF, 1)], sem_par)
    c3.wait()
    c4.wait()
    c5.wait()

    lanes = lax.iota(jnp.int32, L)
    vrow = jnp.minimum(lanes, VOCAB - 1)
    fcwb = params_v[pl.ds(_FCW_OFF, L)]
    s1 = jnp.take_along_axis(fcwb, _full(2 * DIM), axis=0)
    s2 = jnp.zeros((L,), jnp.float32)
    for d in range(DIM):
        col = plsc.load_gather(params_v, [vrow * DIM + d])
        wa = jnp.take_along_axis(fcwb, _full(d), axis=0)
        wb = jnp.take_along_axis(fcwb, _full(DIM + d), axis=0)
        s1 = s1 + col * wa
        s2 = s2 + col * wb
    s1n = -s1
    s2n = -s2

    for v1 in range(VOCAB):
        b1 = jnp.take_along_axis(s1n, _full(v1), axis=0)
        tab_v[pl.ds(v1 * L, L)] = 1.0 / (1.0 + jnp.exp(b1 + s2n))

    c1.wait()
    c2.wait()

    for j in range(B_PER_W // L):
        sl = pl.ds(j * L, L)
        idx = (w1_v[sl] << 4) + w2_v[sl]
        out_v[sl] = plsc.load_gather(tab_v, [idx])

    pltpu.sync_copy(out_v, out_hbm.at[pl.ds(base, B_PER_W)])


@jax.jit
def kernel(w1, w2, embedding, fc_w, fc_b):
    mesh = plsc.VectorSubcoreMesh(core_axis_name="c", subcore_axis_name="s",
                                  num_cores=NC, num_subcores=NS)
    run = pl.kernel(
        _body,
        out_type=jax.ShapeDtypeStruct((BATCH,), jnp.float32),
        mesh=mesh,
        scratch_types=[
            pltpu.VMEM((B_PER_W,), jnp.int32),
            pltpu.VMEM((B_PER_W,), jnp.int32),
            pltpu.VMEM((B_PER_W,), jnp.float32),
            pltpu.VMEM((128,), jnp.float32),
            pltpu.VMEM((VOCAB * L,), jnp.float32),
            pltpu.SemaphoreType.DMA,
            pltpu.SemaphoreType.DMA,
        ],
        compiler_params=pltpu.CompilerParams(needs_layout_passes=False),
        name="similarity_sc",
    )
    return run(w1, w2, embedding.reshape(-1), fc_w.reshape(-1), fc_b)


# speedup vs baseline: 1.0487x; 1.0400x over previous
import jax
import jax.numpy as jnp
from jax import lax
from jax.experimental import pallas as pl
from jax.experimental.pallas import tpu as pltpu
from jax.experimental.pallas import tpu_sc as plsc

VOCAB = 10
DIM = 4
BATCH = 16384
NC = 2
NS = 16
L = 16
NW = NC * NS
B_PER_W = BATCH // NW

_EMB_OFF = 0
_FCW_OFF = 40
_FCB_OFF = 48


def _full(val):
    return jnp.full((L,), val, jnp.int32)


def _body(w1_hbm, w2_hbm, embf_hbm, fcw_hbm, fcb_hbm, out_hbm,
          w1_v, w2_v, out_v, params_v, tab_v, sem_idx, sem_par):
    wid = lax.axis_index("s") * NC + lax.axis_index("c")
    base = wid * B_PER_W

    c1 = pltpu.async_copy(w1_hbm.at[pl.ds(base, B_PER_W)], w1_v, sem_idx)
    c2 = pltpu.async_copy(w2_hbm.at[pl.ds(base, B_PER_W)], w2_v, sem_idx)
    c3 = pltpu.async_copy(embf_hbm, params_v.at[pl.ds(_EMB_OFF, VOCAB * DIM)],
                          sem_par)
    c4 = pltpu.async_copy(fcw_hbm, params_v.at[pl.ds(_FCW_OFF, 2 * DIM)],
                          sem_par)
    c5 = pltpu.async_copy(fcb_hbm, params_v.at[pl.ds(_FCB_OFF, 1)], sem_par)
    c3.wait()
    c4.wait()
    c5.wait()

    lanes = lax.iota(jnp.int32, L)
    vrow = jnp.minimum(lanes, VOCAB - 1)
    fcwb = params_v[pl.ds(_FCW_OFF, L)]
    s1 = jnp.take_along_axis(fcwb, _full(2 * DIM), axis=0)
    s2 = jnp.zeros((L,), jnp.float32)
    for d in range(DIM):
        col = plsc.load_gather(params_v, [vrow * DIM + d])
        wa = jnp.take_along_axis(fcwb, _full(d), axis=0)
        wb = jnp.take_along_axis(fcwb, _full(DIM + d), axis=0)
        s1 = s1 + col * wa
        s2 = s2 + col * wb
    s1n = -s1
    s2n = -s2

    for v1 in range(VOCAB):
        b1 = jnp.take_along_axis(s1n, _full(v1), axis=0)
        tab_v[pl.ds(v1 * L, L)] = 1.0 / (1.0 + jnp.exp(b1 + s2n))

    c1.wait()
    c2.wait()

    @plsc.parallel_loop(0, B_PER_W, step=L, unroll=4)
    def _hot(off):
        sl = pl.ds(off, L)
        idx = (w1_v[sl] << 4) + w2_v[sl]
        out_v[sl] = plsc.load_gather(tab_v, [idx])

    pltpu.sync_copy(out_v, out_hbm.at[pl.ds(base, B_PER_W)])


@jax.jit
def kernel(w1, w2, embedding, fc_w, fc_b):
    mesh = plsc.VectorSubcoreMesh(core_axis_name="c", subcore_axis_name="s",
                                  num_cores=NC, num_subcores=NS)
    run = pl.kernel(
        _body,
        out_type=jax.ShapeDtypeStruct((BATCH,), jnp.float32),
        mesh=mesh,
        scratch_types=[
            pltpu.VMEM((B_PER_W,), jnp.int32),
            pltpu.VMEM((B_PER_W,), jnp.int32),
            pltpu.VMEM((B_PER_W,), jnp.float32),
            pltpu.VMEM((128,), jnp.float32),
            pltpu.VMEM((VOCAB * L,), jnp.float32),
            pltpu.SemaphoreType.DMA,
            pltpu.SemaphoreType.DMA,
        ],
        compiler_params=pltpu.CompilerParams(needs_layout_passes=False),
        name="similarity_sc",
    )
    return run(w1, w2, embedding.reshape(-1), fc_w.reshape(-1), fc_b)
